# baseline (device time: 144100 ns/iter reference)
import jax
import jax.numpy as jnp
from jax import lax
from jax.experimental import pallas as pl
from jax.experimental.pallas import tpu as pltpu

N_DEV = 16
N_TOK = 2048
D_IN = 512
D_OUT = 1024
N_EXP = 64
E_PER = N_EXP // N_DEV
CAP = 25
CAP_PAD = 32
BLK = E_PER * CAP_PAD
T_PER = N_TOK // N_DEV


def _body(x_ref, ew_ref, psrc_ref, pdst_ref, out_ref,
          comm_ref, send_sems, recv_sems):
    my = lax.axis_index("i")
    left = lax.rem(my - 1 + N_DEV, N_DEV)
    right = lax.rem(my + 1, N_DEV)

    barrier_sem = pltpu.get_barrier_semaphore()
    for nbr in (left, right):
        pl.semaphore_signal(barrier_sem, inc=1, device_id=(nbr,),
                            device_id_type=pl.DeviceIdType.MESH)
    pl.semaphore_wait(barrier_sem, 2)

    x_sel = jnp.dot(psrc_ref[...], x_ref[...],
                    preferred_element_type=jnp.float32)
    for s in range(E_PER):
        comm_ref[0, s * CAP_PAD:(s + 1) * CAP_PAD, :] = jnp.dot(
            x_sel[s * CAP_PAD:(s + 1) * CAP_PAD, :], ew_ref[s],
            preferred_element_type=jnp.float32)

    for h in range(N_DEV):
        if h < N_DEV - 1:
            rdma = pltpu.make_async_remote_copy(
                src_ref=comm_ref.at[h],
                dst_ref=comm_ref.at[h + 1],
                send_sem=send_sems.at[h],
                recv_sem=recv_sems.at[h],
                device_id=(right,),
                device_id_type=pl.DeviceIdType.MESH,
            )
            rdma.start()
        contrib = jnp.dot(pdst_ref[h], comm_ref[h],
                          preferred_element_type=jnp.float32)
        if h == 0:
            out_ref[...] = contrib
        else:
            out_ref[...] += contrib
        if h < N_DEV - 1:
            rdma.wait()


def kernel(x, router_W, route_idx, expert_W):
    del router_W
    my = lax.axis_index("i")
    route = route_idx[:, 0]

    onehot = (route[:, None]
              == jnp.arange(N_EXP, dtype=jnp.int32)[None, :]).astype(jnp.int32)
    excl = jnp.cumsum(onehot, axis=0) - onehot
    pos = jnp.sum(excl * onehot, axis=1)

    r_ids = jnp.arange(BLK, dtype=jnp.int32)
    exp_of_r = E_PER * my + r_ids // CAP_PAD
    c_of_r = r_ids % CAP_PAD
    p_src = ((route[None, :] == exp_of_r[:, None])
             & (pos[None, :] == c_of_r[:, None])
             & (c_of_r[:, None] < CAP)).astype(jnp.float32)

    route_g = lax.dynamic_slice(route, (my * T_PER,), (T_PER,))
    pos_g = lax.dynamic_slice(pos, (my * T_PER,), (T_PER,))
    k_g = route_g // E_PER
    h_g = lax.rem(my - k_g + N_DEV, N_DEV)
    gcol = h_g * BLK + lax.rem(route_g, E_PER) * CAP_PAD + pos_g
    p_flat = ((gcol[:, None]
               == jnp.arange(N_DEV * BLK, dtype=jnp.int32)[None, :])
              & (pos_g[:, None] < CAP)).astype(jnp.float32)
    p_dst = p_flat.reshape(T_PER, N_DEV, BLK).transpose(1, 0, 2)

    return pl.pallas_call(
        _body,
        out_shape=jax.ShapeDtypeStruct((T_PER, D_OUT), jnp.float32),
        in_specs=[pl.BlockSpec(memory_space=pltpu.VMEM)] * 4,
        out_specs=pl.BlockSpec(memory_space=pltpu.VMEM),
        scratch_shapes=[
            pltpu.VMEM((N_DEV, BLK, D_OUT), jnp.float32),
            pltpu.SemaphoreType.DMA((N_DEV - 1,)),
            pltpu.SemaphoreType.DMA((N_DEV - 1,)),
        ],
        compiler_params=pltpu.CompilerParams(collective_id=0),
    )(x, expert_W, p_src, p_dst)


# device time: 96590 ns/iter; 1.4919x vs baseline; 1.4919x over previous
import jax
import jax.numpy as jnp
from jax import lax
from jax.experimental import pallas as pl
from jax.experimental.pallas import tpu as pltpu

N_DEV = 16
N_TOK = 2048
D_IN = 512
D_OUT = 1024
N_EXP = 64
E_PER = N_EXP // N_DEV
CAP = 25
CAP_PAD = 32
BLK = E_PER * CAP_PAD
T_PER = N_TOK // N_DEV
H = N_DEV // 2

_PERM = (0, 4, 8, 12, 13, 9, 5, 1, 2, 6, 10, 14, 15, 11, 7, 3)
_RINGPOS = [0] * N_DEV
for _p, _d in enumerate(_PERM):
    _RINGPOS[_d] = _p


def _body(x_ref, ew_ref, psrc_ref, pdst_ref, nbr_ref, out_ref,
          comm_ref, send_r, recv_r, send_l, recv_l):
    right = nbr_ref[0]
    left = nbr_ref[1]

    barrier_sem = pltpu.get_barrier_semaphore()
    for nbr in (left, right):
        pl.semaphore_signal(barrier_sem, inc=1, device_id=(nbr,),
                            device_id_type=pl.DeviceIdType.MESH)
    pl.semaphore_wait(barrier_sem, 2)

    x_sel = jnp.dot(psrc_ref[...], x_ref[...],
                    preferred_element_type=jnp.float32)
    for s in range(E_PER):
        comm_ref[0, s * CAP_PAD:(s + 1) * CAP_PAD, :] = jnp.dot(
            x_sel[s * CAP_PAD:(s + 1) * CAP_PAD, :], ew_ref[s],
            preferred_element_type=jnp.float32)

    for t in range(H + 1):
        rdma_r = rdma_l = None
        if t < H:
            rdma_r = pltpu.make_async_remote_copy(
                src_ref=comm_ref.at[t],
                dst_ref=comm_ref.at[t + 1],
                send_sem=send_r.at[t],
                recv_sem=recv_r.at[t],
                device_id=(right,),
                device_id_type=pl.DeviceIdType.MESH,
            )
            rdma_r.start()
        if t < H - 1:
            rdma_l = pltpu.make_async_remote_copy(
                src_ref=comm_ref.at[(N_DEV - t) % N_DEV],
                dst_ref=comm_ref.at[N_DEV - 1 - t],
                send_sem=send_l.at[t],
                recv_sem=recv_l.at[t],
                device_id=(left,),
                device_id_type=pl.DeviceIdType.MESH,
            )
            rdma_l.start()
        if t == 0:
            out_ref[...] = jnp.dot(pdst_ref[0], comm_ref[0],
                                   preferred_element_type=jnp.float32)
        else:
            out_ref[...] += jnp.dot(pdst_ref[t], comm_ref[t],
                                    preferred_element_type=jnp.float32)
            if t < H:
                out_ref[...] += jnp.dot(pdst_ref[N_DEV - t],
                                        comm_ref[N_DEV - t],
                                        preferred_element_type=jnp.float32)
        if rdma_r is not None:
            rdma_r.wait()
        if rdma_l is not None:
            rdma_l.wait()


def kernel(x, router_W, route_idx, expert_W):
    del router_W
    my = lax.axis_index("i")
    route = route_idx[:, 0]

    perm = jnp.array(_PERM, dtype=jnp.int32)
    ringpos = jnp.array(_RINGPOS, dtype=jnp.int32)
    my_pos = ringpos[my]
    right_dev = perm[lax.rem(my_pos + 1, N_DEV)]
    left_dev = perm[lax.rem(my_pos - 1 + N_DEV, N_DEV)]
    nbrs = jnp.stack([right_dev, left_dev]).astype(jnp.int32)

    onehot = (route[:, None]
              == jnp.arange(N_EXP, dtype=jnp.int32)[None, :]).astype(jnp.int32)
    excl = jnp.cumsum(onehot, axis=0) - onehot
    pos = jnp.sum(excl * onehot, axis=1)

    r_ids = jnp.arange(BLK, dtype=jnp.int32)
    exp_of_r = E_PER * my + r_ids // CAP_PAD
    c_of_r = r_ids % CAP_PAD
    p_src = ((route[None, :] == exp_of_r[:, None])
             & (pos[None, :] == c_of_r[:, None])
             & (c_of_r[:, None] < CAP)).astype(jnp.float32)

    route_g = lax.dynamic_slice(route, (my * T_PER,), (T_PER,))
    pos_g = lax.dynamic_slice(pos, (my * T_PER,), (T_PER,))
    k_g = route_g // E_PER
    h_g = lax.rem(my_pos - ringpos[k_g] + N_DEV, N_DEV)
    gcol = h_g * BLK + lax.rem(route_g, E_PER) * CAP_PAD + pos_g
    p_flat = ((gcol[:, None]
               == jnp.arange(N_DEV * BLK, dtype=jnp.int32)[None, :])
              & (pos_g[:, None] < CAP)).astype(jnp.float32)
    p_dst = p_flat.reshape(T_PER, N_DEV, BLK).transpose(1, 0, 2)

    return pl.pallas_call(
        _body,
        out_shape=jax.ShapeDtypeStruct((T_PER, D_OUT), jnp.float32),
        in_specs=[pl.BlockSpec(memory_space=pltpu.VMEM)] * 4
        + [pl.BlockSpec(memory_space=pltpu.SMEM)],
        out_specs=pl.BlockSpec(memory_space=pltpu.VMEM),
        scratch_shapes=[
            pltpu.VMEM((N_DEV, BLK, D_OUT), jnp.float32),
            pltpu.SemaphoreType.DMA((H,)),
            pltpu.SemaphoreType.DMA((H,)),
            pltpu.SemaphoreType.DMA((H - 1,)),
            pltpu.SemaphoreType.DMA((H - 1,)),
        ],
        compiler_params=pltpu.CompilerParams(collective_id=0),
    )(x, expert_W, p_src, p_dst, nbrs)


# device time: 80404 ns/iter; 1.7922x vs baseline; 1.2013x over previous
import jax
import jax.numpy as jnp
from jax import lax
from jax.experimental import pallas as pl
from jax.experimental.pallas import tpu as pltpu

N_DEV = 16
N_TOK = 2048
D_IN = 512
D_OUT = 1024
N_EXP = 64
E_PER = N_EXP // N_DEV
CAP = 25
CAP_PAD = 32
BLK = E_PER * CAP_PAD
T_PER = N_TOK // N_DEV
H = N_DEV // 2

_PERM = (0, 4, 8, 12, 13, 9, 5, 1, 2, 6, 10, 14, 15, 11, 7, 3)
_RINGPOS = [0] * N_DEV
for _p, _d in enumerate(_PERM):
    _RINGPOS[_d] = _p


def _body(x_ref, ew_ref, psrc_ref, pdst_ref, nbr_ref, out_ref,
          comm_ref, send_r, recv_r, send_l, recv_l):
    right = nbr_ref[0]
    left = nbr_ref[1]

    barrier_sem = pltpu.get_barrier_semaphore()
    for nbr in (left, right):
        pl.semaphore_signal(barrier_sem, inc=1, device_id=(nbr,),
                            device_id_type=pl.DeviceIdType.MESH)
    pl.semaphore_wait(barrier_sem, 2)

    x_sel = jnp.dot(psrc_ref[...], x_ref[...],
                    preferred_element_type=jnp.float32)
    for s in range(E_PER):
        comm_ref[0, s * CAP_PAD:(s + 1) * CAP_PAD, :] = jnp.dot(
            x_sel[s * CAP_PAD:(s + 1) * CAP_PAD, :], ew_ref[s],
            preferred_element_type=jnp.float32)

    for t in range(H + 1):
        rdma_r = rdma_l = None
        if t < H:
            rdma_r = pltpu.make_async_remote_copy(
                src_ref=comm_ref.at[t],
                dst_ref=comm_ref.at[t + 1],
                send_sem=send_r.at[t],
                recv_sem=recv_r.at[t],
                device_id=(right,),
                device_id_type=pl.DeviceIdType.MESH,
            )
            rdma_r.start()
        if t < H - 1:
            rdma_l = pltpu.make_async_remote_copy(
                src_ref=comm_ref.at[(N_DEV - t) % N_DEV],
                dst_ref=comm_ref.at[N_DEV - 1 - t],
                send_sem=send_l.at[t],
                recv_sem=recv_l.at[t],
                device_id=(left,),
                device_id_type=pl.DeviceIdType.MESH,
            )
            rdma_l.start()
        if t == 0:
            out_ref[...] = jnp.dot(pdst_ref[0], comm_ref[0],
                                   preferred_element_type=jnp.float32)
        else:
            out_ref[...] += jnp.dot(pdst_ref[t], comm_ref[t],
                                    preferred_element_type=jnp.float32)
            if t < H:
                out_ref[...] += jnp.dot(pdst_ref[N_DEV - t],
                                        comm_ref[N_DEV - t],
                                        preferred_element_type=jnp.float32)
        if rdma_r is not None:
            rdma_r.wait()
        if rdma_l is not None:
            rdma_l.wait()


def kernel(x, router_W, route_idx, expert_W):
    del router_W
    my = lax.axis_index("i")
    route = route_idx[:, 0]

    perm = jnp.array(_PERM, dtype=jnp.int32)
    ringpos = jnp.array(_RINGPOS, dtype=jnp.int32)
    my_pos = ringpos[my]
    right_dev = perm[lax.rem(my_pos + 1, N_DEV)]
    left_dev = perm[lax.rem(my_pos - 1 + N_DEV, N_DEV)]
    nbrs = jnp.stack([right_dev, left_dev]).astype(jnp.int32)

    onehot = (route[:, None]
              == jnp.arange(N_EXP, dtype=jnp.int32)[None, :]).astype(jnp.float32)
    blk = onehot.reshape(16, 128, N_EXP)
    tri128 = jnp.tril(jnp.ones((128, 128), jnp.float32), -1)
    intra = jnp.einsum("ij,bjk->bik", tri128, blk,
                       preferred_element_type=jnp.float32)
    sums = blk.sum(axis=1)
    tri16 = jnp.tril(jnp.ones((16, 16), jnp.float32), -1)
    carry = tri16 @ sums
    excl = (intra + carry[:, None, :]).reshape(N_TOK, N_EXP)
    pos = jnp.sum(excl * onehot, axis=1).astype(jnp.int32)

    r_ids = jnp.arange(BLK, dtype=jnp.int32)
    exp_of_r = E_PER * my + r_ids // CAP_PAD
    c_of_r = r_ids % CAP_PAD
    p_src = ((route[None, :] == exp_of_r[:, None])
             & (pos[None, :] == c_of_r[:, None])
             & (c_of_r[:, None] < CAP)).astype(jnp.float32)

    route_g = lax.dynamic_slice(route, (my * T_PER,), (T_PER,))
    pos_g = lax.dynamic_slice(pos, (my * T_PER,), (T_PER,))
    k_g = route_g // E_PER
    h_g = lax.rem(my_pos - ringpos[k_g] + N_DEV, N_DEV)
    gcol = h_g * BLK + lax.rem(route_g, E_PER) * CAP_PAD + pos_g
    p_flat = ((gcol[:, None]
               == jnp.arange(N_DEV * BLK, dtype=jnp.int32)[None, :])
              & (pos_g[:, None] < CAP)).astype(jnp.float32)
    p_dst = p_flat.reshape(T_PER, N_DEV, BLK).transpose(1, 0, 2)

    return pl.pallas_call(
        _body,
        out_shape=jax.ShapeDtypeStruct((T_PER, D_OUT), jnp.float32),
        in_specs=[pl.BlockSpec(memory_space=pltpu.VMEM)] * 4
        + [pl.BlockSpec(memory_space=pltpu.SMEM)],
        out_specs=pl.BlockSpec(memory_space=pltpu.VMEM),
        scratch_shapes=[
            pltpu.VMEM((N_DEV, BLK, D_OUT), jnp.float32),
            pltpu.SemaphoreType.DMA((H,)),
            pltpu.SemaphoreType.DMA((H,)),
            pltpu.SemaphoreType.DMA((H - 1,)),
            pltpu.SemaphoreType.DMA((H - 1,)),
        ],
        compiler_params=pltpu.CompilerParams(collective_id=0),
    )(x, expert_W, p_src, p_dst, nbrs)


# device time: 67859 ns/iter; 2.1235x vs baseline; 1.1849x over previous
import jax
import jax.numpy as jnp
from jax import lax
from jax.experimental import pallas as pl
from jax.experimental.pallas import tpu as pltpu

N_DEV = 16
N_TOK = 2048
D_IN = 512
D_OUT = 1024
N_EXP = 64
E_PER = N_EXP // N_DEV
CAP = 25
CAP_PAD = 32
BLK = E_PER * CAP_PAD
T_PER = N_TOK // N_DEV
H = N_DEV // 2

_PERM = (0, 4, 8, 12, 13, 9, 5, 1, 2, 6, 10, 14, 15, 11, 7, 3)
_RINGPOS = [0] * N_DEV
for _p, _d in enumerate(_PERM):
    _RINGPOS[_d] = _p


N_CHUNK = 4
CH = BLK // N_CHUNK


def _body(x_ref, ew_ref, psrc_ref, pdst_ref, nbr_ref, out_ref,
          comm_ref, send_r, recv_r, send_l, recv_l):
    right = nbr_ref[0]
    left = nbr_ref[1]

    x_sel = jnp.dot(psrc_ref[...], x_ref[...],
                    preferred_element_type=jnp.float32)
    for s in range(E_PER):
        comm_ref[0, s * CAP_PAD:(s + 1) * CAP_PAD, :] = jnp.dot(
            x_sel[s * CAP_PAD:(s + 1) * CAP_PAD, :], ew_ref[s],
            preferred_element_type=jnp.float32)

    barrier_sem = pltpu.get_barrier_semaphore()
    for nbr in (left, right):
        pl.semaphore_signal(barrier_sem, inc=1, device_id=(nbr,),
                            device_id_type=pl.DeviceIdType.MESH)
    pl.semaphore_wait(barrier_sem, 2)

    def mk(src_slot, dst_slot, c, ssem, rsem, dev):
        return pltpu.make_async_remote_copy(
            src_ref=comm_ref.at[src_slot, pl.ds(c * CH, CH)],
            dst_ref=comm_ref.at[dst_slot, pl.ds(c * CH, CH)],
            send_sem=ssem,
            recv_sem=rsem,
            device_id=(dev,),
            device_id_type=pl.DeviceIdType.MESH,
        )

    r_desc = [[None] * N_CHUNK for _ in range(H)]
    l_desc = [[None] * N_CHUNK for _ in range(H - 1)]
    for t in range(H + 1):
        for c in range(N_CHUNK):
            if t > 0:
                r_desc[t - 1][c].wait_recv()
            if t < H:
                d = mk(t, t + 1, c, send_r.at[t, c], recv_r.at[t, c], right)
                d.start()
                r_desc[t][c] = d
        for c in range(N_CHUNK):
            if 0 < t <= H - 1:
                l_desc[t - 1][c].wait_recv()
            if t < H - 1:
                d = mk((N_DEV - t) % N_DEV, N_DEV - 1 - t, c,
                       send_l.at[t, c], recv_l.at[t, c], left)
                d.start()
                l_desc[t][c] = d
        if t == 0:
            out_ref[...] = jnp.dot(pdst_ref[0], comm_ref[0],
                                   preferred_element_type=jnp.float32)
        else:
            out_ref[...] += jnp.dot(pdst_ref[t], comm_ref[t],
                                    preferred_element_type=jnp.float32)
            if t < H:
                out_ref[...] += jnp.dot(pdst_ref[N_DEV - t],
                                        comm_ref[N_DEV - t],
                                        preferred_element_type=jnp.float32)

    for row in r_desc + l_desc:
        for d in row:
            d.wait_send()


def kernel(x, router_W, route_idx, expert_W):
    del router_W
    my = lax.axis_index("i")
    route = route_idx[:, 0]

    perm = jnp.array(_PERM, dtype=jnp.int32)
    ringpos = jnp.array(_RINGPOS, dtype=jnp.int32)
    my_pos = ringpos[my]
    right_dev = perm[lax.rem(my_pos + 1, N_DEV)]
    left_dev = perm[lax.rem(my_pos - 1 + N_DEV, N_DEV)]
    nbrs = jnp.stack([right_dev, left_dev]).astype(jnp.int32)

    onehot = (route[:, None]
              == jnp.arange(N_EXP, dtype=jnp.int32)[None, :]).astype(jnp.float32)
    blk = onehot.reshape(16, 128, N_EXP)
    tri128 = jnp.tril(jnp.ones((128, 128), jnp.float32), -1)
    intra = jnp.einsum("ij,bjk->bik", tri128, blk,
                       preferred_element_type=jnp.float32)
    sums = blk.sum(axis=1)
    tri16 = jnp.tril(jnp.ones((16, 16), jnp.float32), -1)
    carry = tri16 @ sums
    excl = (intra + carry[:, None, :]).reshape(N_TOK, N_EXP)
    pos = jnp.sum(excl * onehot, axis=1).astype(jnp.int32)

    r_ids = jnp.arange(BLK, dtype=jnp.int32)
    exp_of_r = E_PER * my + r_ids // CAP_PAD
    c_of_r = r_ids % CAP_PAD
    p_src = ((route[None, :] == exp_of_r[:, None])
             & (pos[None, :] == c_of_r[:, None])
             & (c_of_r[:, None] < CAP)).astype(jnp.float32)

    route_g = lax.dynamic_slice(route, (my * T_PER,), (T_PER,))
    pos_g = lax.dynamic_slice(pos, (my * T_PER,), (T_PER,))
    k_g = route_g // E_PER
    h_g = lax.rem(my_pos - ringpos[k_g] + N_DEV, N_DEV)
    gcol = h_g * BLK + lax.rem(route_g, E_PER) * CAP_PAD + pos_g
    p_flat = ((gcol[:, None]
               == jnp.arange(N_DEV * BLK, dtype=jnp.int32)[None, :])
              & (pos_g[:, None] < CAP)).astype(jnp.float32)
    p_dst = p_flat.reshape(T_PER, N_DEV, BLK).transpose(1, 0, 2)

    return pl.pallas_call(
        _body,
        out_shape=jax.ShapeDtypeStruct((T_PER, D_OUT), jnp.float32),
        in_specs=[pl.BlockSpec(memory_space=pltpu.VMEM)] * 4
        + [pl.BlockSpec(memory_space=pltpu.SMEM)],
        out_specs=pl.BlockSpec(memory_space=pltpu.VMEM),
        scratch_shapes=[
            pltpu.VMEM((N_DEV, BLK, D_OUT), jnp.float32),
            pltpu.SemaphoreType.DMA((H, N_CHUNK)),
            pltpu.SemaphoreType.DMA((H, N_CHUNK)),
            pltpu.SemaphoreType.DMA((H - 1, N_CHUNK)),
            pltpu.SemaphoreType.DMA((H - 1, N_CHUNK)),
        ],
        compiler_params=pltpu.CompilerParams(collective_id=0),
    )(x, expert_W, p_src, p_dst, nbrs)


# device time: 67475 ns/iter; 2.1356x vs baseline; 1.0057x over previous
import jax
import jax.numpy as jnp
from jax import lax
from jax.experimental import pallas as pl
from jax.experimental.pallas import tpu as pltpu

N_DEV = 16
N_TOK = 2048
D_IN = 512
D_OUT = 1024
N_EXP = 64
E_PER = N_EXP // N_DEV
CAP = 25
CAP_PAD = 32
BLK = E_PER * CAP_PAD
T_PER = N_TOK // N_DEV
H = N_DEV // 2

_PERM = (0, 4, 8, 12, 13, 9, 5, 1, 2, 6, 10, 14, 15, 11, 7, 3)
_RINGPOS = [0] * N_DEV
for _p, _d in enumerate(_PERM):
    _RINGPOS[_d] = _p


N_CHUNK = 4
CH = BLK // N_CHUNK


def _body(x_ref, ew_ref, psrc_ref, pdst_ref, nbr_ref, out_ref,
          comm_ref, send_r, recv_r, send_l, recv_l):
    right = nbr_ref[0]
    left = nbr_ref[1]

    barrier_sem = pltpu.get_barrier_semaphore()
    for nbr in (left, right):
        pl.semaphore_signal(barrier_sem, inc=1, device_id=(nbr,),
                            device_id_type=pl.DeviceIdType.MESH)
    pl.semaphore_wait(barrier_sem, 2)

    def mk(src_slot, dst_slot, c, ssem, rsem, dev):
        return pltpu.make_async_remote_copy(
            src_ref=comm_ref.at[src_slot, pl.ds(c * CH, CH)],
            dst_ref=comm_ref.at[dst_slot, pl.ds(c * CH, CH)],
            send_sem=ssem,
            recv_sem=rsem,
            device_id=(dev,),
            device_id_type=pl.DeviceIdType.MESH,
        )

    r_desc = [[None] * N_CHUNK for _ in range(H)]
    l_desc = [[None] * N_CHUNK for _ in range(H - 1)]

    for s in range(E_PER):
        x_sel_s = jnp.dot(psrc_ref[s * CH:(s + 1) * CH, :], x_ref[...],
                          preferred_element_type=jnp.float32)
        comm_ref[0, s * CH:(s + 1) * CH, :] = jnp.dot(
            x_sel_s, ew_ref[s], preferred_element_type=jnp.float32)
        d = mk(0, 1, s, send_r.at[0, s], recv_r.at[0, s], right)
        d.start()
        r_desc[0][s] = d
        d = mk(0, N_DEV - 1, s, send_l.at[0, s], recv_l.at[0, s], left)
        d.start()
        l_desc[0][s] = d

    out_ref[...] = jnp.dot(pdst_ref[0], comm_ref[0],
                           preferred_element_type=jnp.float32)

    for t in range(1, H + 1):
        for c in range(N_CHUNK):
            r_desc[t - 1][c].wait_recv()
            if t < H:
                d = mk(t, t + 1, c, send_r.at[t, c], recv_r.at[t, c], right)
                d.start()
                r_desc[t][c] = d
        for c in range(N_CHUNK):
            if t <= H - 1:
                l_desc[t - 1][c].wait_recv()
            if t < H - 1:
                d = mk((N_DEV - t) % N_DEV, N_DEV - 1 - t, c,
                       send_l.at[t, c], recv_l.at[t, c], left)
                d.start()
                l_desc[t][c] = d
        out_ref[...] += jnp.dot(pdst_ref[t], comm_ref[t],
                                preferred_element_type=jnp.float32)
        if t < H:
            out_ref[...] += jnp.dot(pdst_ref[N_DEV - t],
                                    comm_ref[N_DEV - t],
                                    preferred_element_type=jnp.float32)

    for row in r_desc + l_desc:
        for d in row:
            d.wait_send()


def kernel(x, router_W, route_idx, expert_W):
    del router_W
    my = lax.axis_index("i")
    route = route_idx[:, 0]

    perm = jnp.array(_PERM, dtype=jnp.int32)
    ringpos = jnp.array(_RINGPOS, dtype=jnp.int32)
    my_pos = ringpos[my]
    right_dev = perm[lax.rem(my_pos + 1, N_DEV)]
    left_dev = perm[lax.rem(my_pos - 1 + N_DEV, N_DEV)]
    nbrs = jnp.stack([right_dev, left_dev]).astype(jnp.int32)

    onehot = (route[:, None]
              == jnp.arange(N_EXP, dtype=jnp.int32)[None, :]).astype(jnp.float32)
    blk = onehot.reshape(16, 128, N_EXP)
    tri128 = jnp.tril(jnp.ones((128, 128), jnp.float32), -1)
    intra = jnp.einsum("ij,bjk->bik", tri128, blk,
                       preferred_element_type=jnp.float32)
    sums = blk.sum(axis=1)
    tri16 = jnp.tril(jnp.ones((16, 16), jnp.float32), -1)
    carry = tri16 @ sums
    excl = (intra + carry[:, None, :]).reshape(N_TOK, N_EXP)
    pos = jnp.sum(excl * onehot, axis=1).astype(jnp.int32)

    r_ids = jnp.arange(BLK, dtype=jnp.int32)
    exp_of_r = E_PER * my + r_ids // CAP_PAD
    c_of_r = r_ids % CAP_PAD
    p_src = ((route[None, :] == exp_of_r[:, None])
             & (pos[None, :] == c_of_r[:, None])
             & (c_of_r[:, None] < CAP)).astype(jnp.float32)

    route_g = lax.dynamic_slice(route, (my * T_PER,), (T_PER,))
    pos_g = lax.dynamic_slice(pos, (my * T_PER,), (T_PER,))
    k_g = route_g // E_PER
    h_g = lax.rem(my_pos - ringpos[k_g] + N_DEV, N_DEV)
    gcol = h_g * BLK + lax.rem(route_g, E_PER) * CAP_PAD + pos_g
    p_flat = ((gcol[:, None]
               == jnp.arange(N_DEV * BLK, dtype=jnp.int32)[None, :])
              & (pos_g[:, None] < CAP)).astype(jnp.float32)
    p_dst = p_flat.reshape(T_PER, N_DEV, BLK).transpose(1, 0, 2)

    return pl.pallas_call(
        _body,
        out_shape=jax.ShapeDtypeStruct((T_PER, D_OUT), jnp.float32),
        in_specs=[pl.BlockSpec(memory_space=pltpu.VMEM)] * 4
        + [pl.BlockSpec(memory_space=pltpu.SMEM)],
        out_specs=pl.BlockSpec(memory_space=pltpu.VMEM),
        scratch_shapes=[
            pltpu.VMEM((N_DEV, BLK, D_OUT), jnp.float32),
            pltpu.SemaphoreType.DMA((H, N_CHUNK)),
            pltpu.SemaphoreType.DMA((H, N_CHUNK)),
            pltpu.SemaphoreType.DMA((H - 1, N_CHUNK)),
            pltpu.SemaphoreType.DMA((H - 1, N_CHUNK)),
        ],
        compiler_params=pltpu.CompilerParams(collective_id=0),
    )(x, expert_W, p_src, p_dst, nbrs)


# device time: 46746 ns/iter; 3.0826x vs baseline; 1.4434x over previous
import jax
import jax.numpy as jnp
from jax import lax
from jax.experimental import pallas as pl
from jax.experimental.pallas import tpu as pltpu

N_DEV = 16
N_TOK = 2048
D_IN = 512
D_OUT = 1024
N_EXP = 64
E_PER = N_EXP // N_DEV
CAP = 25
CAP_PAD = 32
BLK = E_PER * CAP_PAD
T_PER = N_TOK // N_DEV
H = N_DEV // 2

_PERM = (0, 4, 8, 12, 13, 9, 5, 1, 2, 6, 10, 14, 15, 11, 7, 3)
_RINGPOS = [0] * N_DEV
for _p, _d in enumerate(_PERM):
    _RINGPOS[_d] = _p


N_CHUNK = 4
CH = BLK // N_CHUNK


def _body(x_ref, ew_ref, psrc_ref, pdst_ref, nbr_ref, out_ref,
          comm_ref, send_r, recv_r, send_l, recv_l):
    right = nbr_ref[0]
    left = nbr_ref[1]

    barrier_sem = pltpu.get_barrier_semaphore()
    for nbr in (left, right):
        pl.semaphore_signal(barrier_sem, inc=1, device_id=(nbr,),
                            device_id_type=pl.DeviceIdType.MESH)
    pl.semaphore_wait(barrier_sem, 2)

    def mk(src_slot, dst_slot, c, ssem, rsem, dev):
        return pltpu.make_async_remote_copy(
            src_ref=comm_ref.at[src_slot, pl.ds(c * CH, CH)],
            dst_ref=comm_ref.at[dst_slot, pl.ds(c * CH, CH)],
            send_sem=ssem,
            recv_sem=rsem,
            device_id=(dev,),
            device_id_type=pl.DeviceIdType.MESH,
        )

    r_desc = [[None] * N_CHUNK for _ in range(H)]
    l_desc = [[None] * N_CHUNK for _ in range(H - 1)]

    for s in range(E_PER):
        x_sel_s = jnp.dot(psrc_ref[s * CH:(s + 1) * CH, :], x_ref[...],
                          preferred_element_type=jnp.float32)
        comm_ref[0, s * CH:(s + 1) * CH, :] = jnp.dot(
            x_sel_s, ew_ref[s],
            preferred_element_type=jnp.float32).astype(jnp.bfloat16)
        d = mk(0, 1, s, send_r.at[0, s], recv_r.at[0, s], right)
        d.start()
        r_desc[0][s] = d
        d = mk(0, N_DEV - 1, s, send_l.at[0, s], recv_l.at[0, s], left)
        d.start()
        l_desc[0][s] = d

    out_ref[...] = jnp.dot(pdst_ref[0], comm_ref[0],
                           preferred_element_type=jnp.float32)

    for t in range(1, H + 1):
        for c in range(N_CHUNK):
            r_desc[t - 1][c].wait_recv()
            if t < H:
                d = mk(t, t + 1, c, send_r.at[t, c], recv_r.at[t, c], right)
                d.start()
                r_desc[t][c] = d
        for c in range(N_CHUNK):
            if t <= H - 1:
                l_desc[t - 1][c].wait_recv()
            if t < H - 1:
                d = mk((N_DEV - t) % N_DEV, N_DEV - 1 - t, c,
                       send_l.at[t, c], recv_l.at[t, c], left)
                d.start()
                l_desc[t][c] = d
        out_ref[...] += jnp.dot(pdst_ref[t], comm_ref[t],
                                preferred_element_type=jnp.float32)
        if t < H:
            out_ref[...] += jnp.dot(pdst_ref[N_DEV - t],
                                    comm_ref[N_DEV - t],
                                    preferred_element_type=jnp.float32)

    for row in r_desc + l_desc:
        for d in row:
            d.wait_send()


def kernel(x, router_W, route_idx, expert_W):
    del router_W
    my = lax.axis_index("i")
    route = route_idx[:, 0]

    perm = jnp.array(_PERM, dtype=jnp.int32)
    ringpos = jnp.array(_RINGPOS, dtype=jnp.int32)
    my_pos = ringpos[my]
    right_dev = perm[lax.rem(my_pos + 1, N_DEV)]
    left_dev = perm[lax.rem(my_pos - 1 + N_DEV, N_DEV)]
    nbrs = jnp.stack([right_dev, left_dev]).astype(jnp.int32)

    onehot = (route[:, None]
              == jnp.arange(N_EXP, dtype=jnp.int32)[None, :]).astype(jnp.float32)
    blk = onehot.reshape(16, 128, N_EXP)
    tri128 = jnp.tril(jnp.ones((128, 128), jnp.float32), -1)
    intra = jnp.einsum("ij,bjk->bik", tri128, blk,
                       preferred_element_type=jnp.float32)
    sums = blk.sum(axis=1)
    tri16 = jnp.tril(jnp.ones((16, 16), jnp.float32), -1)
    carry = tri16 @ sums
    excl = (intra + carry[:, None, :]).reshape(N_TOK, N_EXP)
    pos = jnp.sum(excl * onehot, axis=1).astype(jnp.int32)

    r_ids = jnp.arange(BLK, dtype=jnp.int32)
    exp_of_r = E_PER * my + r_ids // CAP_PAD
    c_of_r = r_ids % CAP_PAD
    p_src = ((route[None, :] == exp_of_r[:, None])
             & (pos[None, :] == c_of_r[:, None])
             & (c_of_r[:, None] < CAP)).astype(jnp.float32)

    route_g = lax.dynamic_slice(route, (my * T_PER,), (T_PER,))
    pos_g = lax.dynamic_slice(pos, (my * T_PER,), (T_PER,))
    k_g = route_g // E_PER
    h_g = lax.rem(my_pos - ringpos[k_g] + N_DEV, N_DEV)
    gcol = h_g * BLK + lax.rem(route_g, E_PER) * CAP_PAD + pos_g
    p_flat = ((gcol[:, None]
               == jnp.arange(N_DEV * BLK, dtype=jnp.int32)[None, :])
              & (pos_g[:, None] < CAP)).astype(jnp.float32)
    p_dst = p_flat.reshape(T_PER, N_DEV, BLK).transpose(1, 0, 2)
    p_dst = p_dst.astype(jnp.bfloat16)

    return pl.pallas_call(
        _body,
        out_shape=jax.ShapeDtypeStruct((T_PER, D_OUT), jnp.float32),
        in_specs=[pl.BlockSpec(memory_space=pltpu.VMEM)] * 4
        + [pl.BlockSpec(memory_space=pltpu.SMEM)],
        out_specs=pl.BlockSpec(memory_space=pltpu.VMEM),
        scratch_shapes=[
            pltpu.VMEM((N_DEV, BLK, D_OUT), jnp.bfloat16),
            pltpu.SemaphoreType.DMA((H, N_CHUNK)),
            pltpu.SemaphoreType.DMA((H, N_CHUNK)),
            pltpu.SemaphoreType.DMA((H - 1, N_CHUNK)),
            pltpu.SemaphoreType.DMA((H - 1, N_CHUNK)),
        ],
        compiler_params=pltpu.CompilerParams(collective_id=0),
    )(x, expert_W, p_src, p_dst, nbrs)


# device time: 43423 ns/iter; 3.3185x vs baseline; 1.0765x over previous
import jax
import jax.numpy as jnp
from jax import lax
from jax.experimental import pallas as pl
from jax.experimental.pallas import tpu as pltpu

N_DEV = 16
N_TOK = 2048
D_IN = 512
D_OUT = 1024
N_EXP = 64
E_PER = N_EXP // N_DEV
CAP = 25
CAP_PAD = 32
BLK = E_PER * CAP_PAD
T_PER = N_TOK // N_DEV
H = N_DEV // 2
N_CHUNK = 4
CH = BLK // N_CHUNK

_PERM = (0, 4, 8, 12, 13, 9, 5, 1, 2, 6, 10, 14, 15, 11, 7, 3)
_RINGPOS = [0] * N_DEV
for _p, _d in enumerate(_PERM):
    _RINGPOS[_d] = _p


def _body(x_ref, ew_ref, route_ref, nbr_ref, out_ref,
          comm_ref, pos_ref, send_r, recv_r, send_l, recv_l):
    my = lax.axis_index("i")
    right = nbr_ref[0]
    left = nbr_ref[1]

    barrier_sem = pltpu.get_barrier_semaphore()
    for nbr in (left, right):
        pl.semaphore_signal(barrier_sem, inc=1, device_id=(nbr,),
                            device_id_type=pl.DeviceIdType.MESH)

    route_i = route_ref[...]
    oh = (lax.broadcasted_iota(jnp.int32, (N_EXP, N_TOK), 0)
          == route_i).astype(jnp.bfloat16)
    tri = (lax.broadcasted_iota(jnp.int32, (N_TOK, N_TOK), 0)
           < lax.broadcasted_iota(jnp.int32, (N_TOK, N_TOK), 1)
           ).astype(jnp.bfloat16)
    excl = jnp.dot(oh, tri, preferred_element_type=jnp.float32)
    pos_ref[...] = jnp.sum(excl * oh.astype(jnp.float32), axis=0,
                           keepdims=True)
    pos_f = pos_ref[...]

    row_id = lax.broadcasted_iota(jnp.int32, (BLK, N_TOK), 0)
    exp_of_r = my * E_PER + row_id // CAP_PAD
    c_of_r = row_id % CAP_PAD
    psrc = ((exp_of_r == route_i)
            & (c_of_r.astype(jnp.float32) == pos_f)
            & (c_of_r < CAP)).astype(jnp.float32)

    pl.semaphore_wait(barrier_sem, 2)

    def mk(src_slot, dst_slot, c, ssem, rsem, dev):
        return pltpu.make_async_remote_copy(
            src_ref=comm_ref.at[src_slot, pl.ds(c * CH, CH)],
            dst_ref=comm_ref.at[dst_slot, pl.ds(c * CH, CH)],
            send_sem=ssem,
            recv_sem=rsem,
            device_id=(dev,),
            device_id_type=pl.DeviceIdType.MESH,
        )

    r_desc = [[None] * N_CHUNK for _ in range(H)]
    l_desc = [[None] * N_CHUNK for _ in range(H - 1)]

    for s in range(E_PER):
        x_sel_s = jnp.dot(psrc[s * CH:(s + 1) * CH, :], x_ref[...],
                          preferred_element_type=jnp.float32)
        comm_ref[0, s * CH:(s + 1) * CH, :] = jnp.dot(
            x_sel_s, ew_ref[s],
            preferred_element_type=jnp.float32).astype(jnp.bfloat16)
        d = mk(0, 1, s, send_r.at[0, s], recv_r.at[0, s], right)
        d.start()
        r_desc[0][s] = d
        d = mk(0, N_DEV - 1, s, send_l.at[0, s], recv_l.at[0, s], left)
        d.start()
        l_desc[0][s] = d

    route_g = route_ref[:, pl.ds(my * T_PER, T_PER)]
    pos_g = pos_ref[:, pl.ds(my * T_PER, T_PER)]
    k_g = route_g // E_PER
    w_g = lax.rem(k_g, 4)
    z_g = k_g // 4
    rp_g = w_g * 4 + jnp.where(lax.rem(w_g, 2) == 0, z_g, 3 - z_g)
    my_pos = nbr_ref[2]
    slot_g = lax.rem(my_pos - rp_g + N_DEV, N_DEV)
    gcol_f = (slot_g * BLK
              + lax.rem(route_g, E_PER) * CAP_PAD).astype(jnp.float32) + pos_g
    kept = pos_g < CAP

    r_sub = lax.broadcasted_iota(jnp.int32, (BLK, T_PER), 0)

    def contrib(t):
        sel = (gcol_f == (r_sub + t * BLK).astype(jnp.float32)) & kept
        pt = sel.astype(jnp.bfloat16)
        return lax.dot_general(pt, comm_ref[t],
                               (((0,), (0,)), ((), ())),
                               preferred_element_type=jnp.float32)

    out_ref[...] = contrib(0)

    for t in range(1, H + 1):
        for c in range(N_CHUNK):
            r_desc[t - 1][c].wait_recv()
            if t < H:
                d = mk(t, t + 1, c, send_r.at[t, c], recv_r.at[t, c], right)
                d.start()
                r_desc[t][c] = d
        for c in range(N_CHUNK):
            if t <= H - 1:
                l_desc[t - 1][c].wait_recv()
            if t < H - 1:
                d = mk((N_DEV - t) % N_DEV, N_DEV - 1 - t, c,
                       send_l.at[t, c], recv_l.at[t, c], left)
                d.start()
                l_desc[t][c] = d
        out_ref[...] += contrib(t)
        if t < H:
            out_ref[...] += contrib(N_DEV - t)

    for row in r_desc + l_desc:
        for d in row:
            d.wait_send()


def kernel(x, router_W, route_idx, expert_W):
    del router_W
    my = lax.axis_index("i")

    perm = jnp.array(_PERM, dtype=jnp.int32)
    ringpos = jnp.array(_RINGPOS, dtype=jnp.int32)
    my_pos = ringpos[my]
    right_dev = perm[lax.rem(my_pos + 1, N_DEV)]
    left_dev = perm[lax.rem(my_pos - 1 + N_DEV, N_DEV)]
    nbrs = jnp.stack([right_dev, left_dev, my_pos]).astype(jnp.int32)

    route_row = route_idx.reshape(1, N_TOK)

    return pl.pallas_call(
        _body,
        out_shape=jax.ShapeDtypeStruct((T_PER, D_OUT), jnp.float32),
        in_specs=[
            pl.BlockSpec(memory_space=pltpu.VMEM),
            pl.BlockSpec(memory_space=pltpu.VMEM),
            pl.BlockSpec(memory_space=pltpu.VMEM),
            pl.BlockSpec(memory_space=pltpu.SMEM),
        ],
        out_specs=pl.BlockSpec(memory_space=pltpu.VMEM),
        scratch_shapes=[
            pltpu.VMEM((N_DEV, BLK, D_OUT), jnp.bfloat16),
            pltpu.VMEM((1, N_TOK), jnp.float32),
            pltpu.SemaphoreType.DMA((H, N_CHUNK)),
            pltpu.SemaphoreType.DMA((H, N_CHUNK)),
            pltpu.SemaphoreType.DMA((H - 1, N_CHUNK)),
            pltpu.SemaphoreType.DMA((H - 1, N_CHUNK)),
        ],
        compiler_params=pltpu.CompilerParams(collective_id=0),
    )(x, expert_W, route_row, nbrs)


# device time: 40043 ns/iter; 3.5986x vs baseline; 1.0844x over previous
import jax
import jax.numpy as jnp
from jax import lax
from jax.experimental import pallas as pl
from jax.experimental.pallas import tpu as pltpu

N_DEV = 16
N_TOK = 2048
D_IN = 512
D_OUT = 1024
N_EXP = 64
E_PER = N_EXP // N_DEV
CAP = 25
CAP_PAD = 32
BLK = E_PER * CAP_PAD
T_PER = N_TOK // N_DEV
H = N_DEV // 2
N_CHUNK = 8
CH = BLK // N_CHUNK
CPE = CAP_PAD // CH

_PERM = (0, 4, 8, 12, 13, 9, 5, 1, 2, 6, 10, 14, 15, 11, 7, 3)
_RINGPOS = [0] * N_DEV
for _p, _d in enumerate(_PERM):
    _RINGPOS[_d] = _p


def _body(x_ref, ew_ref, route_ref, nbr_ref, out_ref,
          comm_ref, pos_ref, send_r, recv_r, send_l, recv_l):
    my = lax.axis_index("i")
    right = nbr_ref[0]
    left = nbr_ref[1]

    barrier_sem = pltpu.get_barrier_semaphore()
    for nbr in (left, right):
        pl.semaphore_signal(barrier_sem, inc=1, device_id=(nbr,),
                            device_id_type=pl.DeviceIdType.MESH)

    route_i = route_ref[...]
    oh = (lax.broadcasted_iota(jnp.int32, (N_EXP, N_TOK), 0)
          == route_i).astype(jnp.bfloat16)
    tri = (lax.broadcasted_iota(jnp.int32, (N_TOK, N_TOK), 0)
           < lax.broadcasted_iota(jnp.int32, (N_TOK, N_TOK), 1)
           ).astype(jnp.bfloat16)
    excl = jnp.dot(oh, tri, preferred_element_type=jnp.float32)
    pos_ref[...] = jnp.sum(excl * oh.astype(jnp.float32), axis=0,
                           keepdims=True)
    pos_f = pos_ref[...]

    row_id = lax.broadcasted_iota(jnp.int32, (BLK, N_TOK), 0)
    exp_of_r = my * E_PER + row_id // CAP_PAD
    c_of_r = row_id % CAP_PAD
    psrc = ((exp_of_r == route_i)
            & (c_of_r.astype(jnp.float32) == pos_f)
            & (c_of_r < CAP)).astype(jnp.float32)

    pl.semaphore_wait(barrier_sem, 2)

    def mk(src_slot, dst_slot, c, ssem, rsem, dev):
        return pltpu.make_async_remote_copy(
            src_ref=comm_ref.at[src_slot, pl.ds(c * CH, CH)],
            dst_ref=comm_ref.at[dst_slot, pl.ds(c * CH, CH)],
            send_sem=ssem,
            recv_sem=rsem,
            device_id=(dev,),
            device_id_type=pl.DeviceIdType.MESH,
        )

    r_desc = [[None] * N_CHUNK for _ in range(H)]
    l_desc = [[None] * N_CHUNK for _ in range(H - 1)]

    for s in range(E_PER):
        x_sel_s = jnp.dot(psrc[s * CAP_PAD:(s + 1) * CAP_PAD, :], x_ref[...],
                          preferred_element_type=jnp.float32)
        comm_ref[0, s * CAP_PAD:(s + 1) * CAP_PAD, :] = jnp.dot(
            x_sel_s, ew_ref[s],
            preferred_element_type=jnp.float32).astype(jnp.bfloat16)
        for c in range(s * CPE, (s + 1) * CPE):
            d = mk(0, 1, c, send_r.at[0, c], recv_r.at[0, c], right)
            d.start()
            r_desc[0][c] = d
            d = mk(0, N_DEV - 1, c, send_l.at[0, c], recv_l.at[0, c], left)
            d.start()
            l_desc[0][c] = d

    route_g = route_ref[:, pl.ds(my * T_PER, T_PER)]
    pos_g = pos_ref[:, pl.ds(my * T_PER, T_PER)]
    k_g = route_g // E_PER
    w_g = lax.rem(k_g, 4)
    z_g = k_g // 4
    rp_g = w_g * 4 + jnp.where(lax.rem(w_g, 2) == 0, z_g, 3 - z_g)
    my_pos = nbr_ref[2]
    slot_g = lax.rem(my_pos - rp_g + N_DEV, N_DEV)
    gcol_f = (slot_g * BLK
              + lax.rem(route_g, E_PER) * CAP_PAD).astype(jnp.float32) + pos_g
    kept = pos_g < CAP

    r_sub = lax.broadcasted_iota(jnp.int32, (BLK, T_PER), 0)

    def contrib(t):
        sel = (gcol_f == (r_sub + t * BLK).astype(jnp.float32)) & kept
        pt = sel.astype(jnp.bfloat16)
        return lax.dot_general(pt, comm_ref[t],
                               (((0,), (0,)), ((), ())),
                               preferred_element_type=jnp.float32)

    out_ref[...] = contrib(0)

    for t in range(1, H + 1):
        for c in range(N_CHUNK):
            r_desc[t - 1][c].wait_recv()
            if t < H:
                d = mk(t, t + 1, c, send_r.at[t, c], recv_r.at[t, c], right)
                d.start()
                r_desc[t][c] = d
        for c in range(N_CHUNK):
            if t <= H - 1:
                l_desc[t - 1][c].wait_recv()
            if t < H - 1:
                d = mk((N_DEV - t) % N_DEV, N_DEV - 1 - t, c,
                       send_l.at[t, c], recv_l.at[t, c], left)
                d.start()
                l_desc[t][c] = d
        out_ref[...] += contrib(t)
        if t < H:
            out_ref[...] += contrib(N_DEV - t)

    for row in r_desc + l_desc:
        for d in row:
            d.wait_send()


def kernel(x, router_W, route_idx, expert_W):
    del router_W
    my = lax.axis_index("i")

    my_pos = lax.rem(my, 4) * 4 + jnp.where(
        lax.rem(lax.rem(my, 4), 2) == 0, my // 4, 3 - my // 4)

    def dev_at(p):
        p = lax.rem(p + N_DEV, N_DEV)
        w = p // 4
        q = lax.rem(p, 4)
        z = jnp.where(lax.rem(w, 2) == 0, q, 3 - q)
        return z * 4 + w

    nbrs = jnp.stack([dev_at(my_pos + 1), dev_at(my_pos - 1),
                      my_pos]).astype(jnp.int32)

    route_row = route_idx.reshape(1, N_TOK)

    return pl.pallas_call(
        _body,
        out_shape=jax.ShapeDtypeStruct((T_PER, D_OUT), jnp.float32),
        in_specs=[
            pl.BlockSpec(memory_space=pltpu.VMEM),
            pl.BlockSpec(memory_space=pltpu.VMEM),
            pl.BlockSpec(memory_space=pltpu.VMEM),
            pl.BlockSpec(memory_space=pltpu.SMEM),
        ],
        out_specs=pl.BlockSpec(memory_space=pltpu.VMEM),
        scratch_shapes=[
            pltpu.VMEM((N_DEV, BLK, D_OUT), jnp.bfloat16),
            pltpu.VMEM((1, N_TOK), jnp.float32),
            pltpu.SemaphoreType.DMA((H, N_CHUNK)),
            pltpu.SemaphoreType.DMA((H, N_CHUNK)),
            pltpu.SemaphoreType.DMA((H - 1, N_CHUNK)),
            pltpu.SemaphoreType.DMA((H - 1, N_CHUNK)),
        ],
        compiler_params=pltpu.CompilerParams(collective_id=0),
    )(x, expert_W, route_row, nbrs)
